# baseline (speedup 1.0000x reference)
import functools

import jax
import jax.numpy as jnp
from jax import lax
from jax.experimental import pallas as pl
from jax.experimental.pallas import tpu as pltpu
from jax.experimental.pallas import tpu_sc as plsc

BATCH = 16384
EMB = 64
HID = 256

_NC = 2
_NS = 16
_NW = _NC * _NS
_ROWS_PER_W = BATCH // _NW
_G = 16
_NGRP = _ROWS_PER_W // _G


def _sc_gather_body(uid_hbm, uemb_hbm, uout_hbm, idx_u, rows, sem):
    wid = lax.axis_index("s") * _NC + lax.axis_index("c")
    base = wid * _ROWS_PER_W
    pltpu.sync_copy(uid_hbm.at[pl.ds(base, _ROWS_PER_W)], idx_u)

    def issue(g, carry):
        vec = idx_u[pl.ds(g * _G, _G)]
        for k in range(_G):
            r = vec[k]
            i = g * _G + k
            pltpu.async_copy(uemb_hbm.at[pl.ds(r, 1)],
                             rows.at[pl.ds(i, 1)], sem)
        return carry

    lax.fori_loop(0, _NGRP, issue, 0)
    pltpu.make_async_copy(uemb_hbm.at[pl.ds(0, _ROWS_PER_W)], rows,
                          sem).wait()
    pltpu.sync_copy(rows, uout_hbm.at[pl.ds(base, _ROWS_PER_W)])


def _sc_gather(user_id, user_emb):
    mesh = plsc.VectorSubcoreMesh(core_axis_name="c", subcore_axis_name="s")
    scratch = [
        pltpu.VMEM((_ROWS_PER_W,), jnp.int32),
        pltpu.VMEM((_ROWS_PER_W, EMB), jnp.float32),
        pltpu.SemaphoreType.DMA,
    ]
    return pl.kernel(
        _sc_gather_body, mesh=mesh,
        out_type=jax.ShapeDtypeStruct((BATCH, EMB), jnp.float32),
        scratch_types=scratch,
        name="ncf_sc_gather",
    )(user_id, user_emb)


_TC_NSEM = 8
_TC_UNROLL = 8


def _tc_gather_body(iid_smem, iemb_hbm, iout_hbm, *sems):
    def issue(g, carry):
        for k in range(_TC_UNROLL):
            i = g * _TC_UNROLL + k
            r = iid_smem[i]
            pltpu.make_async_copy(
                iemb_hbm.at[pl.ds(r, 1)], iout_hbm.at[pl.ds(i, 1)],
                sems[k % _TC_NSEM]).start()
        return carry

    lax.fori_loop(0, BATCH // _TC_UNROLL, issue, 0)
    per_sem = BATCH // _TC_NSEM
    for s in range(_TC_NSEM):
        pltpu.make_async_copy(iemb_hbm.at[pl.ds(0, per_sem)],
                              iout_hbm.at[pl.ds(0, per_sem)], sems[s]).wait()


def _tc_gather(item_id, item_emb):
    return pl.pallas_call(
        _tc_gather_body,
        grid_spec=pltpu.PrefetchScalarGridSpec(
            num_scalar_prefetch=1,
            grid=(1,),
            in_specs=[pl.BlockSpec(memory_space=pl.ANY)],
            out_specs=pl.BlockSpec(memory_space=pl.ANY),
            scratch_shapes=[pltpu.SemaphoreType.DMA] * _TC_NSEM,
        ),
        out_shape=jax.ShapeDtypeStruct((BATCH, EMB), jnp.float32),
    )(item_id, item_emb)


_BLK = 2048


def _mlp_body(u_ref, i_ref, w1u_ref, w1i_ref, b1_ref, w2_ref, b2_ref, o_ref):
    h = (jnp.dot(u_ref[...], w1u_ref[...], preferred_element_type=jnp.float32)
         + jnp.dot(i_ref[...], w1i_ref[...], preferred_element_type=jnp.float32)
         + b1_ref[...])
    h = jnp.maximum(h, 0.0)
    s = jnp.sum(h * w2_ref[...], axis=1, keepdims=True) + b2_ref[...]
    o_ref[...] = 1.0 / (1.0 + jnp.exp(-s))


def _tc_mlp(uvec, ivec, w1u, w1i, b1r, w2r, b2r):
    grid = (BATCH // _BLK,)
    return pl.pallas_call(
        _mlp_body,
        grid=grid,
        in_specs=[
            pl.BlockSpec((_BLK, EMB), lambda i: (i, 0)),
            pl.BlockSpec((_BLK, EMB), lambda i: (i, 0)),
            pl.BlockSpec((EMB, HID), lambda i: (0, 0)),
            pl.BlockSpec((EMB, HID), lambda i: (0, 0)),
            pl.BlockSpec((1, HID), lambda i: (0, 0)),
            pl.BlockSpec((1, HID), lambda i: (0, 0)),
            pl.BlockSpec((1, 1), lambda i: (0, 0)),
        ],
        out_specs=pl.BlockSpec((_BLK, 1), lambda i: (i, 0)),
        out_shape=jax.ShapeDtypeStruct((BATCH, 1), jnp.float32),
    )(uvec, ivec, w1u, w1i, b1r, w2r, b2r)


def kernel(user_id, item_id, user_emb, item_emb, W1, b1, W2, b2):
    uvec = _sc_gather(user_id.astype(jnp.int32), user_emb)
    ivec = _tc_gather(item_id.astype(jnp.int32), item_emb)
    w1u = W1[:EMB]
    w1i = W1[EMB:]
    b1r = b1.reshape(1, HID)
    w2r = W2.reshape(1, HID)
    b2r = b2.reshape(1, 1)
    return _tc_mlp(uvec, ivec, w1u, w1i, b1r, w2r, b2r)


# speedup vs baseline: 1.0065x; 1.0065x over previous
import functools

import jax
import jax.numpy as jnp
from jax import lax
from jax.experimental import pallas as pl
from jax.experimental.pallas import tpu as pltpu
from jax.experimental.pallas import tpu_sc as plsc

BATCH = 16384
EMB = 64
HID = 256

_NC = 2
_NS = 16
_NW = _NC * _NS
_ROWS_PER_W = BATCH // _NW
_G = 16
_NGRP = _ROWS_PER_W // _G


def _sc_gather_body(uid_hbm, uemb_hbm, uout_hbm, idx_u, rows, sem):
    wid = lax.axis_index("s") * _NC + lax.axis_index("c")
    base = wid * _ROWS_PER_W
    pltpu.sync_copy(uid_hbm.at[pl.ds(base, _ROWS_PER_W)], idx_u)

    def issue(g, carry):
        vec = idx_u[pl.ds(g * _G, _G)]
        for k in range(_G):
            r = vec[k]
            i = g * _G + k
            pltpu.async_copy(uemb_hbm.at[pl.ds(r, 1)],
                             rows.at[pl.ds(i, 1)], sem)
        return carry

    lax.fori_loop(0, _NGRP, issue, 0)
    pltpu.make_async_copy(uemb_hbm.at[pl.ds(0, _ROWS_PER_W)], rows,
                          sem).wait()
    pltpu.sync_copy(rows, uout_hbm.at[pl.ds(base, _ROWS_PER_W)])


def _sc_gather(user_id, user_emb):
    mesh = plsc.VectorSubcoreMesh(core_axis_name="c", subcore_axis_name="s")
    scratch = [
        pltpu.VMEM((_ROWS_PER_W,), jnp.int32),
        pltpu.VMEM((_ROWS_PER_W, EMB), jnp.float32),
        pltpu.SemaphoreType.DMA,
    ]
    return pl.kernel(
        _sc_gather_body, mesh=mesh,
        out_type=jax.ShapeDtypeStruct((BATCH, EMB), jnp.float32),
        scratch_types=scratch,
        cost_estimate=pl.CostEstimate(
            flops=0, bytes_accessed=2 * BATCH * EMB * 4, transcendentals=0),
        name="ncf_sc_gather",
    )(user_id, user_emb)


_TC_NSEM = 16
_TC_UNROLL = 16


def _tc_gather_body(iid_smem, iemb_hbm, iout_hbm, *sems):
    def issue(g, carry):
        for k in range(_TC_UNROLL):
            i = g * _TC_UNROLL + k
            r = iid_smem[i]
            pltpu.make_async_copy(
                iemb_hbm.at[pl.ds(r, 1)], iout_hbm.at[pl.ds(i, 1)],
                sems[k % _TC_NSEM]).start()
        return carry

    lax.fori_loop(0, BATCH // _TC_UNROLL, issue, 0)
    per_sem = BATCH // _TC_NSEM
    for s in range(_TC_NSEM):
        pltpu.make_async_copy(iemb_hbm.at[pl.ds(0, per_sem)],
                              iout_hbm.at[pl.ds(0, per_sem)], sems[s]).wait()


def _tc_gather(item_id, item_emb):
    return pl.pallas_call(
        _tc_gather_body,
        grid_spec=pltpu.PrefetchScalarGridSpec(
            num_scalar_prefetch=1,
            grid=(1,),
            in_specs=[pl.BlockSpec(memory_space=pl.ANY)],
            out_specs=pl.BlockSpec(memory_space=pl.ANY),
            scratch_shapes=[pltpu.SemaphoreType.DMA] * _TC_NSEM,
        ),
        out_shape=jax.ShapeDtypeStruct((BATCH, EMB), jnp.float32),
        cost_estimate=pl.CostEstimate(
            flops=0, bytes_accessed=2 * BATCH * EMB * 4, transcendentals=0),
    )(item_id, item_emb)


_BLK = 2048


def _mlp_body(u_ref, i_ref, w1u_ref, w1i_ref, b1_ref, w2_ref, b2_ref, o_ref):
    h = (jnp.dot(u_ref[...], w1u_ref[...], preferred_element_type=jnp.float32)
         + jnp.dot(i_ref[...], w1i_ref[...], preferred_element_type=jnp.float32)
         + b1_ref[...])
    h = jnp.maximum(h, 0.0)
    s = jnp.sum(h * w2_ref[...], axis=1, keepdims=True) + b2_ref[...]
    o_ref[...] = 1.0 / (1.0 + jnp.exp(-s))


def _tc_mlp(uvec, ivec, w1u, w1i, b1r, w2r, b2r):
    grid = (BATCH // _BLK,)
    return pl.pallas_call(
        _mlp_body,
        grid=grid,
        in_specs=[
            pl.BlockSpec((_BLK, EMB), lambda i: (i, 0)),
            pl.BlockSpec((_BLK, EMB), lambda i: (i, 0)),
            pl.BlockSpec((EMB, HID), lambda i: (0, 0)),
            pl.BlockSpec((EMB, HID), lambda i: (0, 0)),
            pl.BlockSpec((1, HID), lambda i: (0, 0)),
            pl.BlockSpec((1, HID), lambda i: (0, 0)),
            pl.BlockSpec((1, 1), lambda i: (0, 0)),
        ],
        out_specs=pl.BlockSpec((_BLK, 1), lambda i: (i, 0)),
        out_shape=jax.ShapeDtypeStruct((BATCH, 1), jnp.float32),
    )(uvec, ivec, w1u, w1i, b1r, w2r, b2r)


def kernel(user_id, item_id, user_emb, item_emb, W1, b1, W2, b2):
    uvec = _sc_gather(user_id.astype(jnp.int32), user_emb)
    ivec = _tc_gather(item_id.astype(jnp.int32), item_emb)
    w1u = W1[:EMB]
    w1i = W1[EMB:]
    b1r = b1.reshape(1, HID)
    w2r = W2.reshape(1, HID)
    b2r = b2.reshape(1, 1)
    return _tc_mlp(uvec, ivec, w1u, w1i, b1r, w2r, b2r)


# speedup vs baseline: 1.5866x; 1.5763x over previous
import functools

import jax
import jax.numpy as jnp
from jax import lax
from jax.experimental import pallas as pl
from jax.experimental.pallas import tpu as pltpu
from jax.experimental.pallas import tpu_sc as plsc

BATCH = 16384
EMB = 64
HID = 256

_NC = 2
_NS = 16
_NW = _NC * _NS
_ROWS_PER_W = BATCH // _NW
_G = 16
_HALF = _ROWS_PER_W // 2


def _gather_body(uid_hbm, iid_hbm, uemb_hbm, iemb_hbm, uout_hbm, iout_hbm,
                 idx_u, idx_i, buf_a, buf_b, s_ua, s_ub, s_ia, s_ib):
    wid = lax.axis_index("s") * _NC + lax.axis_index("c")
    base = wid * _ROWS_PER_W
    pltpu.sync_copy(uid_hbm.at[pl.ds(base, _ROWS_PER_W)], idx_u)
    pltpu.sync_copy(iid_hbm.at[pl.ds(base, _ROWS_PER_W)], idx_i)

    def issue(tab_hbm, idx_ref, lo, buf, sem):
        def body(g, carry):
            vec = idx_ref[pl.ds(lo + g * _G, _G)]
            for k in range(_G):
                r = vec[k]
                pltpu.async_copy(
                    tab_hbm.at[pl.ds(r, 1)],
                    buf.at[pl.ds(g * _G + k, 1)], sem)
            return carry
        lax.fori_loop(0, _HALF // _G, body, 0)

    def drain(sem, buf):
        pltpu.make_async_copy(uemb_hbm.at[pl.ds(0, _HALF)], buf, sem).wait()

    issue(uemb_hbm, idx_u, 0, buf_a, s_ua)
    issue(uemb_hbm, idx_u, _HALF, buf_b, s_ub)

    drain(s_ua, buf_a)
    pltpu.sync_copy(buf_a, uout_hbm.at[pl.ds(base, _HALF)])
    issue(iemb_hbm, idx_i, 0, buf_a, s_ia)

    drain(s_ub, buf_b)
    pltpu.sync_copy(buf_b, uout_hbm.at[pl.ds(base + _HALF, _HALF)])
    issue(iemb_hbm, idx_i, _HALF, buf_b, s_ib)

    drain(s_ia, buf_a)
    pltpu.sync_copy(buf_a, iout_hbm.at[pl.ds(base, _HALF)])
    drain(s_ib, buf_b)
    pltpu.sync_copy(buf_b, iout_hbm.at[pl.ds(base + _HALF, _HALF)])


def _sc_gather(user_id, item_id, user_emb, item_emb):
    mesh = plsc.VectorSubcoreMesh(core_axis_name="c", subcore_axis_name="s")
    out_type = (
        jax.ShapeDtypeStruct((BATCH, EMB), jnp.float32),
        jax.ShapeDtypeStruct((BATCH, EMB), jnp.float32),
    )
    scratch = [
        pltpu.VMEM((_ROWS_PER_W,), jnp.int32),
        pltpu.VMEM((_ROWS_PER_W,), jnp.int32),
        pltpu.VMEM((_HALF, EMB), jnp.float32),
        pltpu.VMEM((_HALF, EMB), jnp.float32),
        pltpu.SemaphoreType.DMA,
        pltpu.SemaphoreType.DMA,
        pltpu.SemaphoreType.DMA,
        pltpu.SemaphoreType.DMA,
    ]
    return pl.kernel(
        _gather_body, mesh=mesh, out_type=out_type, scratch_types=scratch,
        cost_estimate=pl.CostEstimate(
            flops=0, bytes_accessed=4 * BATCH * EMB * 4, transcendentals=0),
        name="ncf_sc_gather",
    )(user_id, item_id, user_emb, item_emb)


_BLK = 2048


def _mlp_body(u_ref, i_ref, w1u_ref, w1i_ref, b1_ref, w2_ref, b2_ref, o_ref):
    h = (jnp.dot(u_ref[...], w1u_ref[...], preferred_element_type=jnp.float32)
         + jnp.dot(i_ref[...], w1i_ref[...], preferred_element_type=jnp.float32)
         + b1_ref[...])
    h = jnp.maximum(h, 0.0)
    s = jnp.sum(h * w2_ref[...], axis=1, keepdims=True) + b2_ref[...]
    o_ref[...] = 1.0 / (1.0 + jnp.exp(-s))


def _tc_mlp(uvec, ivec, w1u, w1i, b1r, w2r, b2r):
    grid = (BATCH // _BLK,)
    return pl.pallas_call(
        _mlp_body,
        grid=grid,
        in_specs=[
            pl.BlockSpec((_BLK, EMB), lambda i: (i, 0)),
            pl.BlockSpec((_BLK, EMB), lambda i: (i, 0)),
            pl.BlockSpec((EMB, HID), lambda i: (0, 0)),
            pl.BlockSpec((EMB, HID), lambda i: (0, 0)),
            pl.BlockSpec((1, HID), lambda i: (0, 0)),
            pl.BlockSpec((1, HID), lambda i: (0, 0)),
            pl.BlockSpec((1, 1), lambda i: (0, 0)),
        ],
        out_specs=pl.BlockSpec((_BLK, 1), lambda i: (i, 0)),
        out_shape=jax.ShapeDtypeStruct((BATCH, 1), jnp.float32),
    )(uvec, ivec, w1u, w1i, b1r, w2r, b2r)


def kernel(user_id, item_id, user_emb, item_emb, W1, b1, W2, b2):
    uvec, ivec = _sc_gather(user_id.astype(jnp.int32),
                            item_id.astype(jnp.int32), user_emb, item_emb)
    w1u = W1[:EMB]
    w1i = W1[EMB:]
    b1r = b1.reshape(1, HID)
    w2r = W2.reshape(1, HID)
    b2r = b2.reshape(1, 1)
    return _tc_mlp(uvec, ivec, w1u, w1i, b1r, w2r, b2r)
